# Initial kernel scaffold; baseline (speedup 1.0000x reference)
import jax
import jax.numpy as jnp
from jax.experimental import pallas as pl


def kernel(customer_x, fund_x, edge_index, edge_attr, Wu, bu, Wi, bi, c1_Wl, c1_bl, c1_Wr, c1_br, c1_We, c1_att, c1_bias, c2_Wl, c2_bl, c2_Wr, c2_br, c2_att, c2_bias, p_W1, p_b1, p_W2, p_b2, k_W1, k_b1, k_W2, k_b2):
    raise NotImplementedError("write your pallas kernel here")


# speedup vs baseline: 7.5714x; 7.5714x over previous
import functools

import jax
import jax.numpy as jnp
from jax.experimental import pallas as pl

_H = 4
_C = 32
_HC = _H * _C

_NODE_BLK = 2048
_EDGE_BLK = 8192


def _full(shape):
    nd = len(shape)
    return pl.BlockSpec(shape, lambda i: (0,) * nd)


def _rows(blk, ncols):
    return pl.BlockSpec((blk, ncols), lambda i: (i, 0))


def _user_prep_kernel(x_ref, Wu_ref, bu_ref, Wl_ref, bl_ref, Wr_ref, br_ref,
                      xl_ref, xr_ref):
    u = jnp.dot(x_ref[...], Wu_ref[...], preferred_element_type=jnp.float32)
    u = u + bu_ref[...]
    xl_ref[...] = jnp.dot(u, Wl_ref[...],
                          preferred_element_type=jnp.float32) + bl_ref[...]
    xr_ref[...] = jnp.dot(u, Wr_ref[...],
                          preferred_element_type=jnp.float32) + br_ref[...]


def _item_prep_kernel(x_ref, Wi_ref, bi_ref, Wr_ref, br_ref, xr_ref):
    v = jnp.dot(x_ref[...], Wi_ref[...], preferred_element_type=jnp.float32)
    v = v + bi_ref[...]
    xr_ref[...] = jnp.dot(v, Wr_ref[...],
                          preferred_element_type=jnp.float32) + br_ref[...]


def _edge_logits_ea_kernel(xls_ref, xrd_ref, ea_ref, We_ref, M_ref, out_ref):
    x = xls_ref[...] + xrd_ref[...]
    x = x + jnp.dot(ea_ref[...], We_ref[...],
                    preferred_element_type=jnp.float32)
    x = jnp.maximum(x, 0.2 * x)
    out_ref[...] = jnp.dot(x, M_ref[...], preferred_element_type=jnp.float32)


def _edge_logits_kernel(xls_ref, xrd_ref, M_ref, out_ref):
    x = xls_ref[...] + xrd_ref[...]
    x = jnp.maximum(x, 0.2 * x)
    out_ref[...] = jnp.dot(x, M_ref[...], preferred_element_type=jnp.float32)


def _edge_msg_kernel(xls_ref, alpha_ref, St_ref, out_ref):
    a = jnp.dot(alpha_ref[...], St_ref[...],
                preferred_element_type=jnp.float32)
    out_ref[...] = xls_ref[...] * a


def _item_mid_kernel(raw_ref, bias_ref, Wl_ref, bl_ref, xl2_ref):
    h = jax.nn.relu(raw_ref[...] + bias_ref[...])
    xl2_ref[...] = jnp.dot(h, Wl_ref[...],
                           preferred_element_type=jnp.float32) + bl_ref[...]


def _head_kernel(raw_ref, bias_ref, pW1_ref, pb1_ref, pW2_ref, pb2_ref,
                 kW1_ref, kb1_ref, kW2_ref, kb2_ref, scores_ref, z_ref):
    uh = raw_ref[...] + bias_ref[...]
    t = jax.nn.relu(jnp.dot(uh, pW1_ref[...],
                            preferred_element_type=jnp.float32) + pb1_ref[...])
    z_ref[...] = jnp.dot(t, pW2_ref[...],
                         preferred_element_type=jnp.float32) + pb2_ref[...]
    s = jax.nn.relu(jnp.dot(uh, kW1_ref[...],
                            preferred_element_type=jnp.float32) + kb1_ref[...])
    scores_ref[...] = jax.nn.sigmoid(
        jnp.dot(s, kW2_ref[...], preferred_element_type=jnp.float32)
        + kb2_ref[...])


def _seg_softmax(logits, seg, num_segments):
    m = jax.ops.segment_max(logits, seg, num_segments=num_segments)
    m = jnp.where(jnp.isfinite(m), m, 0.0)
    e = jnp.exp(logits - m[seg])
    s = jax.ops.segment_sum(e, seg, num_segments=num_segments)
    return e / (s[seg] + 1e-16)


def kernel(customer_x, fund_x, edge_index, edge_attr, Wu, bu, Wi, bi,
           c1_Wl, c1_bl, c1_Wr, c1_br, c1_We, c1_att, c1_bias,
           c2_Wl, c2_bl, c2_Wr, c2_br, c2_att, c2_bias,
           p_W1, p_b1, p_W2, p_b2, k_W1, k_b1, k_W2, k_b2):
    n_user = customer_x.shape[0]
    n_item = fund_x.shape[0]
    n_edge = edge_index.shape[1]
    f_user = customer_x.shape[1]
    f_item = fund_x.shape[1]
    f_edge = edge_attr.shape[1]

    src = edge_index[0]
    dst = edge_index[1]

    sel = (jnp.arange(_HC)[:, None] // _C
           == jnp.arange(_H)[None, :]).astype(jnp.float32)
    M1 = sel * c1_att.reshape(_HC)[:, None]
    M2 = sel * c2_att.reshape(_HC)[:, None]
    selT = sel.T

    bu2 = bu.reshape(1, -1)
    bi2 = bi.reshape(1, -1)
    c1_bl2 = c1_bl.reshape(1, -1)
    c1_br2 = c1_br.reshape(1, -1)
    c2_bl2 = c2_bl.reshape(1, -1)
    c2_br2 = c2_br.reshape(1, -1)
    c1_bias2 = c1_bias.reshape(1, -1)
    c2_bias2 = c2_bias.reshape(1, -1)
    p_b12 = p_b1.reshape(1, -1)
    p_b22 = p_b2.reshape(1, -1)
    k_b12 = k_b1.reshape(1, -1)
    k_b22 = k_b2.reshape(1, -1)

    f32 = jnp.float32

    xl1, xr2 = pl.pallas_call(
        _user_prep_kernel,
        grid=(pl.cdiv(n_user, _NODE_BLK),),
        in_specs=[
            _rows(_NODE_BLK, f_user),
            _full(Wu.shape), _full(bu2.shape),
            _full(c1_Wl.shape), _full(c1_bl2.shape),
            _full(c2_Wr.shape), _full(c2_br2.shape),
        ],
        out_specs=[_rows(_NODE_BLK, _HC), _rows(_NODE_BLK, _HC)],
        out_shape=[
            jax.ShapeDtypeStruct((n_user, _HC), f32),
            jax.ShapeDtypeStruct((n_user, _HC), f32),
        ],
    )(customer_x, Wu, bu2, c1_Wl, c1_bl2, c2_Wr, c2_br2)

    xr1 = pl.pallas_call(
        _item_prep_kernel,
        grid=(pl.cdiv(n_item, _NODE_BLK),),
        in_specs=[
            _rows(_NODE_BLK, f_item),
            _full(Wi.shape), _full(bi2.shape),
            _full(c1_Wr.shape), _full(c1_br2.shape),
        ],
        out_specs=_rows(_NODE_BLK, _HC),
        out_shape=jax.ShapeDtypeStruct((n_item, _HC), f32),
    )(fund_x, Wi, bi2, c1_Wr, c1_br2)

    xl1_src = jnp.take(xl1, src, axis=0)
    xr1_dst = jnp.take(xr1, dst, axis=0)
    logits1 = pl.pallas_call(
        _edge_logits_ea_kernel,
        grid=(pl.cdiv(n_edge, _EDGE_BLK),),
        in_specs=[
            _rows(_EDGE_BLK, _HC), _rows(_EDGE_BLK, _HC),
            _rows(_EDGE_BLK, f_edge),
            _full(c1_We.shape), _full(M1.shape),
        ],
        out_specs=_rows(_EDGE_BLK, _H),
        out_shape=jax.ShapeDtypeStruct((n_edge, _H), f32),
    )(xl1_src, xr1_dst, edge_attr, c1_We, M1)

    alpha1 = _seg_softmax(logits1, dst, n_item)
    msg1 = pl.pallas_call(
        _edge_msg_kernel,
        grid=(pl.cdiv(n_edge, _EDGE_BLK),),
        in_specs=[
            _rows(_EDGE_BLK, _HC), _rows(_EDGE_BLK, _H), _full(selT.shape),
        ],
        out_specs=_rows(_EDGE_BLK, _HC),
        out_shape=jax.ShapeDtypeStruct((n_edge, _HC), f32),
    )(xl1_src, alpha1, selT)
    raw_item = jax.ops.segment_sum(msg1, dst, num_segments=n_item)

    xl2 = pl.pallas_call(
        _item_mid_kernel,
        grid=(pl.cdiv(n_item, _NODE_BLK),),
        in_specs=[
            _rows(_NODE_BLK, _HC), _full(c1_bias2.shape),
            _full(c2_Wl.shape), _full(c2_bl2.shape),
        ],
        out_specs=_rows(_NODE_BLK, _HC),
        out_shape=jax.ShapeDtypeStruct((n_item, _HC), f32),
    )(raw_item, c1_bias2, c2_Wl, c2_bl2)

    xl2_dst = jnp.take(xl2, dst, axis=0)
    xr2_src = jnp.take(xr2, src, axis=0)
    logits2 = pl.pallas_call(
        _edge_logits_kernel,
        grid=(pl.cdiv(n_edge, _EDGE_BLK),),
        in_specs=[
            _rows(_EDGE_BLK, _HC), _rows(_EDGE_BLK, _HC), _full(M2.shape),
        ],
        out_specs=_rows(_EDGE_BLK, _H),
        out_shape=jax.ShapeDtypeStruct((n_edge, _H), f32),
    )(xl2_dst, xr2_src, M2)

    alpha2 = _seg_softmax(logits2, src, n_user)
    msg2 = pl.pallas_call(
        _edge_msg_kernel,
        grid=(pl.cdiv(n_edge, _EDGE_BLK),),
        in_specs=[
            _rows(_EDGE_BLK, _HC), _rows(_EDGE_BLK, _H), _full(selT.shape),
        ],
        out_specs=_rows(_EDGE_BLK, _HC),
        out_shape=jax.ShapeDtypeStruct((n_edge, _HC), f32),
    )(xl2_dst, alpha2, selT)
    raw_user = jax.ops.segment_sum(msg2, src, num_segments=n_user)

    scores, z = pl.pallas_call(
        _head_kernel,
        grid=(pl.cdiv(n_user, _NODE_BLK),),
        in_specs=[
            _rows(_NODE_BLK, _HC), _full(c2_bias2.shape),
            _full(p_W1.shape), _full(p_b12.shape),
            _full(p_W2.shape), _full(p_b22.shape),
            _full(k_W1.shape), _full(k_b12.shape),
            _full(k_W2.shape), _full(k_b22.shape),
        ],
        out_specs=[_rows(_NODE_BLK, 1), _rows(_NODE_BLK, _HC)],
        out_shape=[
            jax.ShapeDtypeStruct((n_user, 1), f32),
            jax.ShapeDtypeStruct((n_user, _HC), f32),
        ],
    )(raw_user, c2_bias2, p_W1, p_b12, p_W2, p_b22,
      k_W1, k_b12, k_W2, k_b22)

    return (scores, z)


# speedup vs baseline: 13.6303x; 1.8002x over previous
import functools

import jax
import jax.numpy as jnp
from jax.experimental import pallas as pl

_H = 4
_C = 32
_HC = _H * _C

_NODE_BLK = 2048
_EDGE_BLK = 8192


def _full(shape):
    nd = len(shape)
    return pl.BlockSpec(shape, lambda i: (0,) * nd)


def _rows(blk, ncols):
    return pl.BlockSpec((blk, ncols), lambda i: (i, 0))


def _user_prep_kernel(x_ref, Wu_ref, bu_ref, Wl_ref, bl_ref, Wr_ref, br_ref,
                      xl_ref, xr_ref):
    u = jnp.dot(x_ref[...], Wu_ref[...], preferred_element_type=jnp.float32)
    u = u + bu_ref[...]
    xl_ref[...] = jnp.dot(u, Wl_ref[...],
                          preferred_element_type=jnp.float32) + bl_ref[...]
    xr_ref[...] = jnp.dot(u, Wr_ref[...],
                          preferred_element_type=jnp.float32) + br_ref[...]


def _item_prep_kernel(x_ref, Wi_ref, bi_ref, Wr_ref, br_ref, xr_ref):
    v = jnp.dot(x_ref[...], Wi_ref[...], preferred_element_type=jnp.float32)
    v = v + bi_ref[...]
    xr_ref[...] = jnp.dot(v, Wr_ref[...],
                          preferred_element_type=jnp.float32) + br_ref[...]


def _edge_logits_ea_kernel(xls_ref, xrd_ref, ea_ref, We_ref, M_ref, out_ref):
    x = xls_ref[...] + xrd_ref[...]
    x = x + jnp.dot(ea_ref[...], We_ref[...],
                    preferred_element_type=jnp.float32)
    x = jnp.maximum(x, 0.2 * x)
    out_ref[...] = jnp.dot(x, M_ref[...], preferred_element_type=jnp.float32)


def _edge_logits_kernel(xls_ref, xrd_ref, M_ref, out_ref):
    x = xls_ref[...] + xrd_ref[...]
    x = jnp.maximum(x, 0.2 * x)
    out_ref[...] = jnp.dot(x, M_ref[...], preferred_element_type=jnp.float32)


def _edge_fused_kernel(xls_ref, logits_ref, gmax_ref, St_ref, out_ref):
    e = jnp.exp(logits_ref[...] - gmax_ref[...])
    a = jnp.dot(e, St_ref[...], preferred_element_type=jnp.float32)
    out_ref[:, :_HC] = xls_ref[...] * a
    out_ref[:, _HC:] = e


def _item_mid_kernel(seg_ref, St_ref, bias_ref, Wl_ref, bl_ref, xl2_ref):
    s = jnp.dot(seg_ref[:, _HC:], St_ref[...],
                preferred_element_type=jnp.float32)
    h = jax.nn.relu(seg_ref[:, :_HC] / (s + 1e-16) + bias_ref[...])
    xl2_ref[...] = jnp.dot(h, Wl_ref[...],
                           preferred_element_type=jnp.float32) + bl_ref[...]


def _head_kernel(seg_ref, St_ref, bias_ref, pW1_ref, pb1_ref, pW2_ref,
                 pb2_ref, kW1_ref, kb1_ref, kW2_ref, kb2_ref,
                 scores_ref, z_ref):
    s = jnp.dot(seg_ref[:, _HC:], St_ref[...],
                preferred_element_type=jnp.float32)
    uh = seg_ref[:, :_HC] / (s + 1e-16) + bias_ref[...]
    t = jax.nn.relu(jnp.dot(uh, pW1_ref[...],
                            preferred_element_type=jnp.float32) + pb1_ref[...])
    z_ref[...] = jnp.dot(t, pW2_ref[...],
                         preferred_element_type=jnp.float32) + pb2_ref[...]
    s = jax.nn.relu(jnp.dot(uh, kW1_ref[...],
                            preferred_element_type=jnp.float32) + kb1_ref[...])
    scores_ref[...] = jax.nn.sigmoid(
        jnp.dot(s, kW2_ref[...], preferred_element_type=jnp.float32)
        + kb2_ref[...])


def kernel(customer_x, fund_x, edge_index, edge_attr, Wu, bu, Wi, bi,
           c1_Wl, c1_bl, c1_Wr, c1_br, c1_We, c1_att, c1_bias,
           c2_Wl, c2_bl, c2_Wr, c2_br, c2_att, c2_bias,
           p_W1, p_b1, p_W2, p_b2, k_W1, k_b1, k_W2, k_b2):
    n_user = customer_x.shape[0]
    n_item = fund_x.shape[0]
    n_edge = edge_index.shape[1]
    f_user = customer_x.shape[1]
    f_item = fund_x.shape[1]
    f_edge = edge_attr.shape[1]

    src = edge_index[0]
    dst = edge_index[1]

    sel = (jnp.arange(_HC)[:, None] // _C
           == jnp.arange(_H)[None, :]).astype(jnp.float32)
    M1 = sel * c1_att.reshape(_HC)[:, None]
    M2 = sel * c2_att.reshape(_HC)[:, None]
    selT = sel.T

    bu2 = bu.reshape(1, -1)
    bi2 = bi.reshape(1, -1)
    c1_bl2 = c1_bl.reshape(1, -1)
    c1_br2 = c1_br.reshape(1, -1)
    c2_bl2 = c2_bl.reshape(1, -1)
    c2_br2 = c2_br.reshape(1, -1)
    c1_bias2 = c1_bias.reshape(1, -1)
    c2_bias2 = c2_bias.reshape(1, -1)
    p_b12 = p_b1.reshape(1, -1)
    p_b22 = p_b2.reshape(1, -1)
    k_b12 = k_b1.reshape(1, -1)
    k_b22 = k_b2.reshape(1, -1)

    f32 = jnp.float32

    xl1, xr2 = pl.pallas_call(
        _user_prep_kernel,
        grid=(pl.cdiv(n_user, _NODE_BLK),),
        in_specs=[
            _rows(_NODE_BLK, f_user),
            _full(Wu.shape), _full(bu2.shape),
            _full(c1_Wl.shape), _full(c1_bl2.shape),
            _full(c2_Wr.shape), _full(c2_br2.shape),
        ],
        out_specs=[_rows(_NODE_BLK, _HC), _rows(_NODE_BLK, _HC)],
        out_shape=[
            jax.ShapeDtypeStruct((n_user, _HC), f32),
            jax.ShapeDtypeStruct((n_user, _HC), f32),
        ],
    )(customer_x, Wu, bu2, c1_Wl, c1_bl2, c2_Wr, c2_br2)

    xr1 = pl.pallas_call(
        _item_prep_kernel,
        grid=(pl.cdiv(n_item, _NODE_BLK),),
        in_specs=[
            _rows(_NODE_BLK, f_item),
            _full(Wi.shape), _full(bi2.shape),
            _full(c1_Wr.shape), _full(c1_br2.shape),
        ],
        out_specs=_rows(_NODE_BLK, _HC),
        out_shape=jax.ShapeDtypeStruct((n_item, _HC), f32),
    )(fund_x, Wi, bi2, c1_Wr, c1_br2)

    xl1_src = jnp.take(xl1, src, axis=0)
    xr1_dst = jnp.take(xr1, dst, axis=0)
    logits1 = pl.pallas_call(
        _edge_logits_ea_kernel,
        grid=(pl.cdiv(n_edge, _EDGE_BLK),),
        in_specs=[
            _rows(_EDGE_BLK, _HC), _rows(_EDGE_BLK, _HC),
            _rows(_EDGE_BLK, f_edge),
            _full(c1_We.shape), _full(M1.shape),
        ],
        out_specs=_rows(_EDGE_BLK, _H),
        out_shape=jax.ShapeDtypeStruct((n_edge, _H), f32),
    )(xl1_src, xr1_dst, edge_attr, c1_We, M1)

    gmax1 = jnp.max(logits1, axis=0, keepdims=True)
    fused1 = pl.pallas_call(
        _edge_fused_kernel,
        grid=(pl.cdiv(n_edge, _EDGE_BLK),),
        in_specs=[
            _rows(_EDGE_BLK, _HC), _rows(_EDGE_BLK, _H),
            _full((1, _H)), _full(selT.shape),
        ],
        out_specs=_rows(_EDGE_BLK, _HC + _H),
        out_shape=jax.ShapeDtypeStruct((n_edge, _HC + _H), f32),
    )(xl1_src, logits1, gmax1, selT)
    seg_item = jax.ops.segment_sum(fused1, dst, num_segments=n_item)

    xl2 = pl.pallas_call(
        _item_mid_kernel,
        grid=(pl.cdiv(n_item, _NODE_BLK),),
        in_specs=[
            _rows(_NODE_BLK, _HC + _H), _full(selT.shape),
            _full(c1_bias2.shape),
            _full(c2_Wl.shape), _full(c2_bl2.shape),
        ],
        out_specs=_rows(_NODE_BLK, _HC),
        out_shape=jax.ShapeDtypeStruct((n_item, _HC), f32),
    )(seg_item, selT, c1_bias2, c2_Wl, c2_bl2)

    xl2_dst = jnp.take(xl2, dst, axis=0)
    xr2_src = jnp.take(xr2, src, axis=0)
    logits2 = pl.pallas_call(
        _edge_logits_kernel,
        grid=(pl.cdiv(n_edge, _EDGE_BLK),),
        in_specs=[
            _rows(_EDGE_BLK, _HC), _rows(_EDGE_BLK, _HC), _full(M2.shape),
        ],
        out_specs=_rows(_EDGE_BLK, _H),
        out_shape=jax.ShapeDtypeStruct((n_edge, _H), f32),
    )(xl2_dst, xr2_src, M2)

    gmax2 = jnp.max(logits2, axis=0, keepdims=True)
    fused2 = pl.pallas_call(
        _edge_fused_kernel,
        grid=(pl.cdiv(n_edge, _EDGE_BLK),),
        in_specs=[
            _rows(_EDGE_BLK, _HC), _rows(_EDGE_BLK, _H),
            _full((1, _H)), _full(selT.shape),
        ],
        out_specs=_rows(_EDGE_BLK, _HC + _H),
        out_shape=jax.ShapeDtypeStruct((n_edge, _HC + _H), f32),
    )(xl2_dst, logits2, gmax2, selT)
    seg_user = jax.ops.segment_sum(fused2, src, num_segments=n_user)

    scores, z = pl.pallas_call(
        _head_kernel,
        grid=(pl.cdiv(n_user, _NODE_BLK),),
        in_specs=[
            _rows(_NODE_BLK, _HC + _H), _full(selT.shape),
            _full(c2_bias2.shape),
            _full(p_W1.shape), _full(p_b12.shape),
            _full(p_W2.shape), _full(p_b22.shape),
            _full(k_W1.shape), _full(k_b12.shape),
            _full(k_W2.shape), _full(k_b22.shape),
        ],
        out_specs=[_rows(_NODE_BLK, 1), _rows(_NODE_BLK, _HC)],
        out_shape=[
            jax.ShapeDtypeStruct((n_user, 1), f32),
            jax.ShapeDtypeStruct((n_user, _HC), f32),
        ],
    )(seg_user, selT, c2_bias2, p_W1, p_b12, p_W2, p_b22,
      k_W1, k_b12, k_W2, k_b22)

    return (scores, z)
